# baseline (speedup 1.0000x reference)
import jax
import jax.numpy as jnp
from jax import lax
from jax.experimental import pallas as pl
from jax.experimental.pallas import tpu as pltpu

_LOG2E = 1.4426950408889634
_CH = 256


def _gat_kernel(inp_ref, w_ref, a_ref, adj_hbm, out_ref,
                adj_buf, sem, h_ref, hb_ref, f1_ref, f2_ref, g1_ref, g2_ref):
    n = h_ref.shape[0]
    nch = n // _CH

    def start_copy(c, slot):
        pltpu.make_async_copy(
            adj_hbm.at[pl.ds(c * _CH, _CH), :],
            adj_buf.at[slot],
            sem.at[slot],
        ).start()

    start_copy(0, 0)

    h = jnp.dot(inp_ref[...], w_ref[...], preferred_element_type=jnp.float32)
    h_ref[...] = h
    hb_ref[...] = h.astype(jnp.bfloat16)
    d = h.shape[1]
    f = lax.dot_general(h, a_ref[:, :d], (((1,), (1,)), ((), ())),
                        preferred_element_type=jnp.float32)
    g = lax.dot_general(a_ref[:, d:], h, (((1,), (1,)), ((), ())),
                        preferred_element_type=jnp.float32)
    f1_ref[...] = f * (-_LOG2E)
    f2_ref[...] = f * (-0.01 * _LOG2E)
    g1_ref[...] = g * (-_LOG2E)
    g2_ref[...] = g * (-0.01 * _LOG2E)

    for c in range(nch):
        slot = c % 2
        if c + 1 < nch:
            start_copy(c + 1, (c + 1) % 2)

        pltpu.make_async_copy(
            adj_hbm.at[pl.ds(c * _CH, _CH), :],
            adj_buf.at[slot],
            sem.at[slot],
        ).wait()

        rows = pl.ds(c * _CH, _CH)
        s1 = f1_ref[rows, :] + g1_ref[...]
        s2 = f2_ref[rows, :] + g2_ref[...]
        e = jnp.exp2(jnp.minimum(s1, s2))
        e = jnp.where(adj_buf[slot] != 0, e, 0.0)
        rowsum = jnp.sum(e, axis=1, keepdims=True)
        hp = jnp.dot(e.astype(jnp.bfloat16), hb_ref[...],
                     preferred_element_type=jnp.float32)
        hp = hp / rowsum
        out_ref[rows, :] = jnp.where(hp > 0.0, hp, jnp.exp(hp) - 1.0)


def kernel(input, adj, W, a):
    n, d_in = input.shape
    d_out = W.shape[1]
    return pl.pallas_call(
        _gat_kernel,
        in_specs=[
            pl.BlockSpec((n, d_in), lambda: (0, 0)),
            pl.BlockSpec((d_in, d_out), lambda: (0, 0)),
            pl.BlockSpec((1, 2 * d_out), lambda: (0, 0)),
            pl.BlockSpec(memory_space=pl.ANY),
        ],
        out_specs=pl.BlockSpec((n, d_out), lambda: (0, 0)),
        out_shape=jax.ShapeDtypeStruct((n, d_out), jnp.float32),
        scratch_shapes=[
            pltpu.VMEM((2, _CH, n), jnp.int32),
            pltpu.SemaphoreType.DMA((2,)),
            pltpu.VMEM((n, d_out), jnp.float32),
            pltpu.VMEM((n, d_out), jnp.bfloat16),
            pltpu.VMEM((n, 1), jnp.float32),
            pltpu.VMEM((n, 1), jnp.float32),
            pltpu.VMEM((1, n), jnp.float32),
            pltpu.VMEM((1, n), jnp.float32),
        ],
    )(input, W, a, adj)


# speedup vs baseline: 1.2672x; 1.2672x over previous
import jax
import jax.numpy as jnp
from jax import lax
from jax.experimental import pallas as pl
from jax.experimental.pallas import tpu as pltpu


_LOG2E = 1.4426950408889634


def _gat_kernel(inp_ref, w_ref, a_ref, adj_ref, out_ref,
                h_ref, hb_ref, f1_ref, f2_ref, g1_ref, g2_ref):
    i = pl.program_id(0)

    @pl.when(i == 0)
    def _():
        h = jnp.dot(inp_ref[...], w_ref[...], preferred_element_type=jnp.float32)
        h_ref[...] = h
        hb_ref[...] = h.astype(jnp.bfloat16)
        d = h.shape[1]
        f = lax.dot_general(
            h, a_ref[:, :d], (((1,), (1,)), ((), ())),
            preferred_element_type=jnp.float32)
        g = lax.dot_general(
            a_ref[:, d:], h, (((1,), (1,)), ((), ())),
            preferred_element_type=jnp.float32)
        f1_ref[...] = f * (-_LOG2E)
        f2_ref[...] = f * (-0.01 * _LOG2E)
        g1_ref[...] = g * (-_LOG2E)
        g2_ref[...] = g * (-0.01 * _LOG2E)

    blk = out_ref.shape[0]
    rows = pl.ds(i * blk, blk)
    s1 = f1_ref[rows, :] + g1_ref[...]
    s2 = f2_ref[rows, :] + g2_ref[...]
    e = jnp.exp2(jnp.minimum(s1, s2))
    e = lax.bitcast_convert_type(
        lax.bitcast_convert_type(e, jnp.int32) * adj_ref[...], jnp.float32)
    rowsum = jnp.sum(e, axis=1, keepdims=True)
    hp = jnp.dot(e.astype(jnp.bfloat16), hb_ref[...],
                 preferred_element_type=jnp.float32)
    hp = hp / rowsum
    out_ref[...] = jnp.where(hp > 0.0, hp, jnp.exp(hp) - 1.0)


def kernel(input, adj, W, a):
    n, d_in = input.shape
    d_out = W.shape[1]
    blk = 512
    return pl.pallas_call(
        _gat_kernel,
        grid=(n // blk,),
        in_specs=[
            pl.BlockSpec((n, d_in), lambda i: (0, 0)),
            pl.BlockSpec((d_in, d_out), lambda i: (0, 0)),
            pl.BlockSpec((1, 2 * d_out), lambda i: (0, 0)),
            pl.BlockSpec((blk, n), lambda i: (i, 0)),
        ],
        out_specs=pl.BlockSpec((blk, d_out), lambda i: (i, 0)),
        out_shape=jax.ShapeDtypeStruct((n, d_out), jnp.float32),
        scratch_shapes=[
            pltpu.VMEM((n, d_out), jnp.float32),
            pltpu.VMEM((n, d_out), jnp.bfloat16),
            pltpu.VMEM((n, 1), jnp.float32),
            pltpu.VMEM((n, 1), jnp.float32),
            pltpu.VMEM((1, n), jnp.float32),
            pltpu.VMEM((1, n), jnp.float32),
        ],
    )(input, W, a, adj)


# speedup vs baseline: 1.2710x; 1.0030x over previous
import jax
import jax.numpy as jnp
from jax import lax
from jax.experimental import pallas as pl
from jax.experimental.pallas import tpu as pltpu


_LOG2E = 1.4426950408889634


def _gat_kernel(inp_ref, w_ref, a_ref, adj_ref, out_ref,
                h_ref, hb_ref, f1_ref, f2_ref, g1_ref, g2_ref):
    i = pl.program_id(0)

    @pl.when(i == 0)
    def _():
        h = jnp.dot(inp_ref[...], w_ref[...], preferred_element_type=jnp.float32)
        h_ref[...] = h
        hb_ref[...] = h.astype(jnp.bfloat16)
        d = h.shape[1]
        f = lax.dot_general(
            h, a_ref[:, :d], (((1,), (1,)), ((), ())),
            preferred_element_type=jnp.float32)
        g = lax.dot_general(
            a_ref[:, d:], h, (((1,), (1,)), ((), ())),
            preferred_element_type=jnp.float32)
        f1_ref[...] = f * (-_LOG2E)
        f2_ref[...] = f * (-0.01 * _LOG2E)
        g1_ref[...] = g * (-_LOG2E)
        g2_ref[...] = g * (-0.01 * _LOG2E)

    blk = out_ref.shape[0]
    rows = pl.ds(i * blk, blk)
    s1 = f1_ref[rows, :] + g1_ref[...]
    s2 = f2_ref[rows, :] + g2_ref[...]
    e = jnp.exp2(jnp.minimum(s1, s2))
    e = lax.bitcast_convert_type(
        lax.bitcast_convert_type(e, jnp.int32) * adj_ref[...], jnp.float32)
    rowsum = jnp.sum(e, axis=1, keepdims=True)
    hp = jnp.dot(e, h_ref[...], preferred_element_type=jnp.float32)
    hp = hp / rowsum
    out_ref[...] = jnp.where(hp > 0.0, hp, jnp.exp(hp) - 1.0)


def kernel(input, adj, W, a):
    n, d_in = input.shape
    d_out = W.shape[1]
    blk = 512
    return pl.pallas_call(
        _gat_kernel,
        grid=(n // blk,),
        in_specs=[
            pl.BlockSpec((n, d_in), lambda i: (0, 0)),
            pl.BlockSpec((d_in, d_out), lambda i: (0, 0)),
            pl.BlockSpec((1, 2 * d_out), lambda i: (0, 0)),
            pl.BlockSpec((blk, n), lambda i: (i, 0)),
        ],
        out_specs=pl.BlockSpec((blk, d_out), lambda i: (i, 0)),
        out_shape=jax.ShapeDtypeStruct((n, d_out), jnp.float32),
        scratch_shapes=[
            pltpu.VMEM((n, d_out), jnp.float32),
            pltpu.VMEM((n, d_out), jnp.bfloat16),
            pltpu.VMEM((n, 1), jnp.float32),
            pltpu.VMEM((n, 1), jnp.float32),
            pltpu.VMEM((1, n), jnp.float32),
            pltpu.VMEM((1, n), jnp.float32),
        ],
    )(input, W, a, adj)
